# baseline (speedup 1.0000x reference)
import functools

import jax
import jax.numpy as jnp
from jax import lax
from jax.experimental import pallas as pl
from jax.experimental.pallas import tpu as pltpu
from jax.experimental.pallas import tpu_sc as plsc

_N = 10000
_E = 320000
_F = 128
_FH = _F // 2
_NG = 64
_NCLS = 32

_NC = 2
_NS = 16
_NW = _NC * _NS

_CH = 80
_ET = _E // _NS
_NCHUNK = _ET // _CH
_DCH = 80
_EW = _E // _NW
_DCHUNK = _EW // _DCH

_NP = 10240
_RPT = _NP // _NS
_ZR = 128

_RB = 2000
_NRB = _N // _RB

_f32 = jnp.float32
_bf16 = jnp.bfloat16


def _sc_mesh():
    return plsc.VectorSubcoreMesh(
        core_axis_name="c", subcore_axis_name="s",
        num_cores=_NC, num_subcores=_NS)


def _build_deg():
    @functools.partial(
        pl.kernel,
        out_type=[jax.ShapeDtypeStruct((_NP,), _f32),
                  jax.ShapeDtypeStruct((_NP,), _f32)],
        mesh=_sc_mesh(),
        scratch_types=[
            pltpu.VMEM((_DCHUNK, _DCH), jnp.int32),
            pltpu.VMEM((_DCH,), _f32),
            pltpu.VMEM((_RPT,), _f32),
            pltpu.VMEM_SHARED((_NP,), _f32),
        ],
        compiler_params=pltpu.CompilerParams(use_tc_tiling_on_sc=False),
    )
    def deg_k(dstd, dega, degb, didx, ones_v, zv, deg_sp):
        c = lax.axis_index("c")
        s = lax.axis_index("s")
        w = c * _NS + s
        pltpu.sync_copy(dstd.at[w], didx)
        for k in range(_DCH // 16):
            ones_v[pl.ds(k * 16, 16)] = jnp.ones((16,), _f32)
        for k in range(_RPT // 16):
            zv[pl.ds(k * 16, 16)] = jnp.zeros((16,), _f32)
        pltpu.sync_copy(zv, deg_sp.at[pl.ds(s * _RPT, _RPT)])
        plsc.subcore_barrier()

        def body(i, carry):
            pltpu.sync_copy(ones_v, deg_sp.at[didx.at[i]], add=True)
            return carry
        lax.fori_loop(0, _DCHUNK, body, 0)
        plsc.subcore_barrier()

        @pl.when(c == 0)
        def _():
            pltpu.sync_copy(deg_sp.at[pl.ds(s * _RPT, _RPT)],
                            dega.at[pl.ds(s * _RPT, _RPT)])

        @pl.when(c == 1)
        def _():
            pltpu.sync_copy(deg_sp.at[pl.ds(s * _RPT, _RPT)],
                            degb.at[pl.ds(s * _RPT, _RPT)])

    return deg_k


_NB = 10
_LA = 6
_LI = 8


def _build_agg():
    @functools.partial(
        pl.kernel,
        out_type=jax.ShapeDtypeStruct((_NP, _F), _bf16),
        mesh=_sc_mesh(),
        scratch_types=[
            pltpu.VMEM((_NB, _CH), jnp.int32),
            pltpu.VMEM((_NCHUNK, _CH), jnp.int32),
            pltpu.VMEM((_NB, _CH, _FH), _bf16),
            pltpu.VMEM((_ZR, _FH), _bf16),
            pltpu.VMEM_SHARED((_NP, _FH), _bf16),
            pltpu.SemaphoreType.DMA((_NB,)),
            pltpu.SemaphoreType.DMA((_NB,)),
            pltpu.SemaphoreType.DMA((_NB,)),
        ],
        compiler_params=pltpu.CompilerParams(use_tc_tiling_on_sc=False),
    )
    def agg_k(srcf, dst3, g, u, sidxr, didx, rows, zbuf, u_sp,
              semi, semg, sems):
        c = lax.axis_index("c")
        s = lax.axis_index("s")
        coff = c * _FH
        pltpu.sync_copy(dst3.at[s], didx)

        def _src_slice(i):
            return srcf.at[pl.ds(s * _ET + i * _CH, _CH)]

        def _idx_start(i, b):
            pltpu.async_copy(_src_slice(i), sidxr.at[b], semi.at[b])

        def _idx_wait(i, b):
            pltpu.make_async_copy(
                _src_slice(i), sidxr.at[b], semi.at[b]).wait()

        def _xform(b):
            for k in range(_CH // 16):
                v = sidxr[b, pl.ds(k * 16, 16)]
                sidxr[b, pl.ds(k * 16, 16)] = v + v + c

        def _gather_start(b):
            pltpu.async_copy(g.at[sidxr.at[b]], rows.at[b], semg.at[b])

        def _gather_wait(b):
            pltpu.make_async_copy(
                g.at[sidxr.at[b]], rows.at[b], semg.at[b]).wait()

        def _scatter_start(i, b):
            pltpu.async_copy(rows.at[b], u_sp.at[didx.at[i]], sems.at[b],
                             add=True)

        def _scatter_wait(i, b):
            pltpu.make_async_copy(
                rows.at[b], u_sp.at[didx.at[i]], sems.at[b]).wait()

        def zrow(i, carry):
            for r in range(_FH // 32):
                zbuf[i, pl.ds(r * 32, 32)] = jnp.zeros((32,), _bf16)
            return carry
        lax.fori_loop(0, _ZR, zrow, 0)
        for j in range(_RPT // _ZR):
            pltpu.sync_copy(zbuf, u_sp.at[pl.ds(s * _RPT + j * _ZR, _ZR)])
        plsc.subcore_barrier()

        for j in range(_LI):
            _idx_start(j, j)
        for j in range(_LA):
            _idx_wait(j, j)
            _xform(j)
            _gather_start(j)

        def body(o, carry):
            for b in range(_NB):
                i = o * _NB + b
                _gather_wait(b)
                _scatter_start(i, b)
                j = i + _LA
                bg = (b + _LA) % _NB

                @pl.when(j < _NCHUNK)
                def _():
                    @pl.when(j >= _NB)
                    def _():
                        _scatter_wait(j - _NB, bg)
                    _idx_wait(j, bg)
                    _xform(bg)
                    _gather_start(bg)
                j2 = i + _LI
                b2 = (b + _LI) % _NB

                @pl.when(j2 < _NCHUNK)
                def _():
                    _idx_start(j2, b2)
            return carry
        lax.fori_loop(0, _NCHUNK // _NB, body, 0)

        for b in range(_NB):
            _scatter_wait(_NCHUNK - _NB + b, b)
        plsc.subcore_barrier()

        pltpu.sync_copy(u_sp.at[pl.ds(s * _RPT, _RPT)],
                        u.at[pl.ds(s * _RPT, _RPT), pl.ds(coff, _FH)])

    return agg_k


_DEG = _build_deg()
_AGG = _build_agg()


def _kmm_body(x_ref, w_ref, lin_ref):
    lin_ref[...] = jnp.dot(x_ref[...], w_ref[...],
                           preferred_element_type=_f32)


def _kmm(x, w0):
    return pl.pallas_call(
        _kmm_body,
        grid=(_NRB,),
        in_specs=[
            pl.BlockSpec((_RB, _F), lambda i: (i, 0)),
            pl.BlockSpec((_F, _F), lambda i: (0, 0)),
        ],
        out_specs=pl.BlockSpec((_RB, _F), lambda i: (i, 0)),
        out_shape=jax.ShapeDtypeStruct((_N, _F), _f32),
    )(x, w0)


def _kb_body(lin_ref, degsum_ref, g_ref, dinv_ref):
    dv = lax.rsqrt(degsum_ref[...])
    dinv_ref[...] = dv
    g_ref[...] = (lin_ref[...] * dv).astype(_bf16)


def _kb(lin, degsum):
    return pl.pallas_call(
        _kb_body,
        grid=(_NRB,),
        in_specs=[
            pl.BlockSpec((_RB, _F), lambda i: (i, 0)),
            pl.BlockSpec((_RB, 1), lambda i: (i, 0)),
        ],
        out_specs=[
            pl.BlockSpec((_RB, _F), lambda i: (i, 0)),
            pl.BlockSpec((_RB, 1), lambda i: (i, 0)),
        ],
        out_shape=[
            jax.ShapeDtypeStruct((_N, _F), _bf16),
            jax.ShapeDtypeStruct((_N, 1), _f32),
        ],
    )(lin, degsum)


def _kc_body(g_ref, u_ref, dinv_ref, b_ref, w_ref, gn_ref):
    dv = dinv_ref[...]
    t = u_ref[...].astype(_f32) + g_ref[...].astype(_f32)
    h = jnp.maximum(dv * t + b_ref[...], 0.0)
    gn_ref[...] = (jnp.dot(h, w_ref[...], preferred_element_type=_f32)
                   * dv).astype(_bf16)


def _kc(g, u, dinv, b, w):
    return pl.pallas_call(
        _kc_body,
        grid=(_NRB,),
        in_specs=[
            pl.BlockSpec((_RB, _F), lambda i: (i, 0)),
            pl.BlockSpec((_RB, _F), lambda i: (i, 0)),
            pl.BlockSpec((_RB, 1), lambda i: (i, 0)),
            pl.BlockSpec((1, _F), lambda i: (0, 0)),
            pl.BlockSpec((_F, _F), lambda i: (0, 0)),
        ],
        out_specs=pl.BlockSpec((_RB, _F), lambda i: (i, 0)),
        out_shape=jax.ShapeDtypeStruct((_N, _F), _bf16),
    )(g, u, dinv, b, w)


def _ke_body(g_ref, u_ref, dinv_ref, b_ref, batch_ref, wl_ref, bl_ref,
             out_ref, sums_ref, cnt_ref):
    i = pl.program_id(0)

    @pl.when(i == 0)
    def _():
        sums_ref[...] = jnp.zeros((_NG, _F), _f32)
        cnt_ref[...] = jnp.zeros((_NG, _F), _f32)

    t = u_ref[...].astype(_f32) + g_ref[...].astype(_f32)
    h = jnp.maximum(dinv_ref[...] * t + b_ref[...], 0.0)
    bids = batch_ref[...].reshape(1, _RB)
    onehot = (lax.broadcasted_iota(jnp.int32, (_NG, _RB), 0)
              == bids).astype(_f32)
    sums_ref[...] += jnp.dot(onehot, h, preferred_element_type=_f32)
    cnt_ref[...] += jnp.broadcast_to(
        jnp.sum(onehot, axis=1, keepdims=True), (_NG, _F))

    @pl.when(i == _NRB - 1)
    def _():
        pooled = sums_ref[...] / jnp.maximum(cnt_ref[...], 1.0)
        out_ref[...] = jnp.dot(pooled, wl_ref[...],
                               preferred_element_type=_f32) + bl_ref[...]


def _ke(g, u, dinv, b, batch3, wlin, blin):
    return pl.pallas_call(
        _ke_body,
        grid=(_NRB,),
        in_specs=[
            pl.BlockSpec((_RB, _F), lambda i: (i, 0)),
            pl.BlockSpec((_RB, _F), lambda i: (i, 0)),
            pl.BlockSpec((_RB, 1), lambda i: (i, 0)),
            pl.BlockSpec((1, _F), lambda i: (0, 0)),
            pl.BlockSpec((1, 1, _RB), lambda i: (i, 0, 0)),
            pl.BlockSpec((_F, _NCLS), lambda i: (0, 0)),
            pl.BlockSpec((1, _NCLS), lambda i: (0, 0)),
        ],
        out_specs=pl.BlockSpec((_NG, _NCLS), lambda i: (0, 0)),
        out_shape=jax.ShapeDtypeStruct((_NG, _NCLS), _f32),
        scratch_shapes=[
            pltpu.VMEM((_NG, _F), _f32),
            pltpu.VMEM((_NG, _F), _f32),
        ],
    )(g, u, dinv, b, batch3, wlin, blin)


def kernel(x, edge_index, batch, W0, b0, W1, b1, W2, b2, Wlin, blin):
    srcf = edge_index[0].astype(jnp.int32)
    dst3 = edge_index[1].reshape(_NS, _NCHUNK, _CH).astype(jnp.int32)
    dstd = edge_index[1].reshape(_NW, _DCHUNK, _DCH).astype(jnp.int32)
    batch3 = batch.reshape(_NRB, 1, _RB).astype(jnp.int32)

    dega_p, degb_p = _DEG(dstd)
    lin0 = _kmm(x, W0)
    degsum = (dega_p + degb_p + 1.0)[:_N].reshape(_N, 1)

    g0, dinv = _kb(lin0, degsum)
    u0 = _AGG(srcf, dst3, g0.reshape(2 * _N, _FH))
    g1 = _kc(g0, u0, dinv, b0.reshape(1, _F), W1)
    u1 = _AGG(srcf, dst3, g1.reshape(2 * _N, _FH))
    g2 = _kc(g1, u1, dinv, b1.reshape(1, _F), W2)
    u2 = _AGG(srcf, dst3, g2.reshape(2 * _N, _FH))
    return _ke(g2, u2, dinv, b2.reshape(1, _F), batch3,
               Wlin, blin.reshape(1, _NCLS))


# speedup vs baseline: 1.1334x; 1.1334x over previous
import functools

import jax
import jax.numpy as jnp
from jax import lax
from jax.experimental import pallas as pl
from jax.experimental.pallas import tpu as pltpu
from jax.experimental.pallas import tpu_sc as plsc

_N = 10000
_E = 320000
_F = 128
_NG = 64
_NCLS = 32

_NC = 2
_NS = 16
_NW = _NC * _NS

_CH = 80
_DCH = 80
_EW = _E // _NW
_DCHUNK = _EW // _DCH

_NP = 10240
_RPT = _NP // _NS
_ZR = 128

_RB = 2000
_NRB = _N // _RB

_f32 = jnp.float32
_bf16 = jnp.bfloat16


def _sc_mesh():
    return plsc.VectorSubcoreMesh(
        core_axis_name="c", subcore_axis_name="s",
        num_cores=_NC, num_subcores=_NS)


def _build_deg():
    @functools.partial(
        pl.kernel,
        out_type=[jax.ShapeDtypeStruct((_NP,), _f32),
                  jax.ShapeDtypeStruct((_NP,), _f32)],
        mesh=_sc_mesh(),
        scratch_types=[
            pltpu.VMEM((_DCHUNK, _DCH), jnp.int32),
            pltpu.VMEM((_DCH,), _f32),
            pltpu.VMEM((_RPT,), _f32),
            pltpu.VMEM_SHARED((_NP,), _f32),
        ],
        compiler_params=pltpu.CompilerParams(use_tc_tiling_on_sc=False),
    )
    def deg_k(dstd, dega, degb, didx, ones_v, zv, deg_sp):
        c = lax.axis_index("c")
        s = lax.axis_index("s")
        w = c * _NS + s
        pltpu.sync_copy(dstd.at[w], didx)
        for k in range(_DCH // 16):
            ones_v[pl.ds(k * 16, 16)] = jnp.ones((16,), _f32)
        for k in range(_RPT // 16):
            zv[pl.ds(k * 16, 16)] = jnp.zeros((16,), _f32)
        pltpu.sync_copy(zv, deg_sp.at[pl.ds(s * _RPT, _RPT)])
        plsc.subcore_barrier()

        def body(i, carry):
            pltpu.sync_copy(ones_v, deg_sp.at[didx.at[i]], add=True)
            return carry
        lax.fori_loop(0, _DCHUNK, body, 0)
        plsc.subcore_barrier()

        @pl.when(c == 0)
        def _():
            pltpu.sync_copy(deg_sp.at[pl.ds(s * _RPT, _RPT)],
                            dega.at[pl.ds(s * _RPT, _RPT)])

        @pl.when(c == 1)
        def _():
            pltpu.sync_copy(deg_sp.at[pl.ds(s * _RPT, _RPT)],
                            degb.at[pl.ds(s * _RPT, _RPT)])

    return deg_k


_NB = 10
_LA = 6
_LI = 8
_TCH = _EW // _CH
_TOUT = (_TCH + _NB - 1) // _NB


def _build_agg():
    @functools.partial(
        pl.kernel,
        out_type=[jax.ShapeDtypeStruct((_NP, _F), _bf16),
                  jax.ShapeDtypeStruct((_NP, _F), _bf16)],
        mesh=_sc_mesh(),
        scratch_types=[
            pltpu.VMEM((_NB, _CH), jnp.int32),
            pltpu.VMEM((_TCH, _CH), jnp.int32),
            pltpu.VMEM((_NB, _CH, _F), _bf16),
            pltpu.VMEM((_ZR, _F), _bf16),
            pltpu.VMEM_SHARED((_NP, _F), _bf16),
            pltpu.SemaphoreType.DMA((_NB,)),
            pltpu.SemaphoreType.DMA((_NB,)),
            pltpu.SemaphoreType.DMA((_NB,)),
        ],
        compiler_params=pltpu.CompilerParams(use_tc_tiling_on_sc=False),
    )
    def agg_k(srcf, dstd, g, ua, ub, sidxr, didx, rows, zbuf, u_sp,
              semi, semg, sems):
        c = lax.axis_index("c")
        s = lax.axis_index("s")
        w = c * _NS + s
        pltpu.sync_copy(dstd.at[w], didx)

        def _src_slice(i):
            return srcf.at[pl.ds(w * _EW + i * _CH, _CH)]

        def _idx_start(i, b):
            pltpu.async_copy(_src_slice(i), sidxr.at[b], semi.at[b])

        def _idx_wait(i, b):
            pltpu.make_async_copy(
                _src_slice(i), sidxr.at[b], semi.at[b]).wait()

        def _gather_start(b):
            pltpu.async_copy(g.at[sidxr.at[b]], rows.at[b], semg.at[b])

        def _gather_wait(b):
            pltpu.make_async_copy(
                g.at[sidxr.at[b]], rows.at[b], semg.at[b]).wait()

        def _scatter_start(i, b):
            pltpu.async_copy(rows.at[b], u_sp.at[didx.at[i]], sems.at[b],
                             add=True)

        def _scatter_wait(i, b):
            pltpu.make_async_copy(
                rows.at[b], u_sp.at[didx.at[i]], sems.at[b]).wait()

        def zrow(i, carry):
            for r in range(_F // 32):
                zbuf[i, pl.ds(r * 32, 32)] = jnp.zeros((32,), _bf16)
            return carry
        lax.fori_loop(0, _ZR, zrow, 0)
        for j in range(_RPT // _ZR):
            pltpu.sync_copy(zbuf, u_sp.at[pl.ds(s * _RPT + j * _ZR, _ZR)])
        plsc.subcore_barrier()

        for j in range(_LI):
            _idx_start(j, j)
        for j in range(_LA):
            _idx_wait(j, j)
            _gather_start(j)

        def body(o, carry):
            for b in range(_NB):
                i = o * _NB + b

                @pl.when(i < _TCH)
                def _():
                    _gather_wait(b)
                    _scatter_start(i, b)
                    j = i + _LA
                    bg = (b + _LA) % _NB

                    @pl.when(j < _TCH)
                    def _():
                        @pl.when(j >= _NB)
                        def _():
                            _scatter_wait(j - _NB, bg)
                        _idx_wait(j, bg)
                        _gather_start(bg)
                    j2 = i + _LI
                    b2 = (b + _LI) % _NB

                    @pl.when(j2 < _TCH)
                    def _():
                        _idx_start(j2, b2)
            return carry
        lax.fori_loop(0, _TOUT, body, 0)

        for b in range(_NB):
            last = _TCH - 1 - ((_TCH - 1 - b) % _NB)
            _scatter_wait(last, b)
        plsc.subcore_barrier()

        @pl.when(c == 0)
        def _():
            pltpu.sync_copy(u_sp.at[pl.ds(s * _RPT, _RPT)],
                            ua.at[pl.ds(s * _RPT, _RPT)])

        @pl.when(c == 1)
        def _():
            pltpu.sync_copy(u_sp.at[pl.ds(s * _RPT, _RPT)],
                            ub.at[pl.ds(s * _RPT, _RPT)])

    return agg_k


_DEG = _build_deg()
_AGG = _build_agg()


def _kmm_body(x_ref, w_ref, lin_ref):
    lin_ref[...] = jnp.dot(x_ref[...], w_ref[...],
                           preferred_element_type=_f32)


def _kmm(x, w0):
    return pl.pallas_call(
        _kmm_body,
        grid=(_NRB,),
        in_specs=[
            pl.BlockSpec((_RB, _F), lambda i: (i, 0)),
            pl.BlockSpec((_F, _F), lambda i: (0, 0)),
        ],
        out_specs=pl.BlockSpec((_RB, _F), lambda i: (i, 0)),
        out_shape=jax.ShapeDtypeStruct((_N, _F), _f32),
    )(x, w0)


def _kb_body(lin_ref, degsum_ref, g_ref, dinv_ref):
    dv = lax.rsqrt(degsum_ref[...])
    dinv_ref[...] = dv
    g_ref[...] = (lin_ref[...] * dv).astype(_bf16)


def _kb(lin, degsum):
    return pl.pallas_call(
        _kb_body,
        grid=(_NRB,),
        in_specs=[
            pl.BlockSpec((_RB, _F), lambda i: (i, 0)),
            pl.BlockSpec((_RB, 1), lambda i: (i, 0)),
        ],
        out_specs=[
            pl.BlockSpec((_RB, _F), lambda i: (i, 0)),
            pl.BlockSpec((_RB, 1), lambda i: (i, 0)),
        ],
        out_shape=[
            jax.ShapeDtypeStruct((_N, _F), _bf16),
            jax.ShapeDtypeStruct((_N, 1), _f32),
        ],
    )(lin, degsum)


def _kc_body(g_ref, ua_ref, ub_ref, dinv_ref, b_ref, w_ref, gn_ref):
    dv = dinv_ref[...]
    t = (ua_ref[...].astype(_f32) + ub_ref[...].astype(_f32)
         + g_ref[...].astype(_f32))
    h = jnp.maximum(dv * t + b_ref[...], 0.0)
    gn_ref[...] = (jnp.dot(h, w_ref[...], preferred_element_type=_f32)
                   * dv).astype(_bf16)


def _kc(g, ua, ub, dinv, b, w):
    return pl.pallas_call(
        _kc_body,
        grid=(_NRB,),
        in_specs=[
            pl.BlockSpec((_RB, _F), lambda i: (i, 0)),
            pl.BlockSpec((_RB, _F), lambda i: (i, 0)),
            pl.BlockSpec((_RB, _F), lambda i: (i, 0)),
            pl.BlockSpec((_RB, 1), lambda i: (i, 0)),
            pl.BlockSpec((1, _F), lambda i: (0, 0)),
            pl.BlockSpec((_F, _F), lambda i: (0, 0)),
        ],
        out_specs=pl.BlockSpec((_RB, _F), lambda i: (i, 0)),
        out_shape=jax.ShapeDtypeStruct((_N, _F), _bf16),
    )(g, ua, ub, dinv, b, w)


def _ke_body(g_ref, ua_ref, ub_ref, dinv_ref, b_ref, batch_ref, wl_ref,
             bl_ref, out_ref, sums_ref, cnt_ref):
    i = pl.program_id(0)

    @pl.when(i == 0)
    def _():
        sums_ref[...] = jnp.zeros((_NG, _F), _f32)
        cnt_ref[...] = jnp.zeros((_NG, _F), _f32)

    t = (ua_ref[...].astype(_f32) + ub_ref[...].astype(_f32)
         + g_ref[...].astype(_f32))
    h = jnp.maximum(dinv_ref[...] * t + b_ref[...], 0.0)
    bids = batch_ref[...].reshape(1, _RB)
    onehot = (lax.broadcasted_iota(jnp.int32, (_NG, _RB), 0)
              == bids).astype(_f32)
    sums_ref[...] += jnp.dot(onehot, h, preferred_element_type=_f32)
    cnt_ref[...] += jnp.broadcast_to(
        jnp.sum(onehot, axis=1, keepdims=True), (_NG, _F))

    @pl.when(i == _NRB - 1)
    def _():
        pooled = sums_ref[...] / jnp.maximum(cnt_ref[...], 1.0)
        out_ref[...] = jnp.dot(pooled, wl_ref[...],
                               preferred_element_type=_f32) + bl_ref[...]


def _ke(g, ua, ub, dinv, b, batch3, wlin, blin):
    return pl.pallas_call(
        _ke_body,
        grid=(_NRB,),
        in_specs=[
            pl.BlockSpec((_RB, _F), lambda i: (i, 0)),
            pl.BlockSpec((_RB, _F), lambda i: (i, 0)),
            pl.BlockSpec((_RB, _F), lambda i: (i, 0)),
            pl.BlockSpec((_RB, 1), lambda i: (i, 0)),
            pl.BlockSpec((1, _F), lambda i: (0, 0)),
            pl.BlockSpec((1, 1, _RB), lambda i: (i, 0, 0)),
            pl.BlockSpec((_F, _NCLS), lambda i: (0, 0)),
            pl.BlockSpec((1, _NCLS), lambda i: (0, 0)),
        ],
        out_specs=pl.BlockSpec((_NG, _NCLS), lambda i: (0, 0)),
        out_shape=jax.ShapeDtypeStruct((_NG, _NCLS), _f32),
        scratch_shapes=[
            pltpu.VMEM((_NG, _F), _f32),
            pltpu.VMEM((_NG, _F), _f32),
        ],
    )(g, ua, ub, dinv, b, batch3, wlin, blin)


def kernel(x, edge_index, batch, W0, b0, W1, b1, W2, b2, Wlin, blin):
    srcf = edge_index[0].astype(jnp.int32)
    dstd = edge_index[1].reshape(_NW, _DCHUNK, _DCH).astype(jnp.int32)
    batch3 = batch.reshape(_NRB, 1, _RB).astype(jnp.int32)

    dega_p, degb_p = _DEG(dstd)
    lin0 = _kmm(x, W0)
    degsum = (dega_p + degb_p + 1.0)[:_N].reshape(_N, 1)

    g0, dinv = _kb(lin0, degsum)
    ua0, ub0 = _AGG(srcf, dstd, g0)
    g1 = _kc(g0, ua0, ub0, dinv, b0.reshape(1, _F), W1)
    ua1, ub1 = _AGG(srcf, dstd, g1)
    g2 = _kc(g1, ua1, ub1, dinv, b1.reshape(1, _F), W2)
    ua2, ub2 = _AGG(srcf, dstd, g2)
    return _ke(g2, ua2, ub2, dinv, b2.reshape(1, _F), batch3,
               Wlin, blin.reshape(1, _NCLS))
